# Initial kernel scaffold; baseline (speedup 1.0000x reference)
import jax
import jax.numpy as jnp
from jax.experimental import pallas as pl


def kernel(x, gate_w, w_gate_up, w_down):
    raise NotImplementedError("write your pallas kernel here")


# speedup vs baseline: 1.3485x; 1.3485x over previous
import jax
import jax.numpy as jnp
from jax.experimental import pallas as pl

E = 8
H = 768
F = 2048
TT = 256
LB_W = 0.01


def _gate_body(x_ref, gw_ref, comb_ref, aux_ref):
    x = x_ref[...]
    gw = gw_ref[...]
    logits = jax.lax.dot_general(
        x, gw, (((1,), (1,)), ((), ())), preferred_element_type=jnp.float32)
    m = jnp.max(logits, axis=-1, keepdims=True)
    ex = jnp.exp(logits - m)
    probs = ex / jnp.sum(ex, axis=-1, keepdims=True)

    iota = jax.lax.broadcasted_iota(jnp.int32, probs.shape, 1)
    p1 = jnp.max(probs, axis=-1, keepdims=True)
    idx1 = jnp.min(jnp.where(probs == p1, iota, E), axis=-1, keepdims=True)
    oh1 = (iota == idx1)
    masked = jnp.where(oh1, -jnp.inf, probs)
    p2 = jnp.max(masked, axis=-1, keepdims=True)
    idx2 = jnp.min(jnp.where(masked == p2, iota, E), axis=-1, keepdims=True)
    oh2 = (iota == idx2)

    denom = p1 + p2 + 1e-9
    oh1f = oh1.astype(jnp.float32)
    oh2f = oh2.astype(jnp.float32)
    comb_ref[...] = (p1 / denom) * oh1f + (p2 / denom) * oh2f

    s = jnp.float32(probs.shape[0])
    f = jnp.sum(oh1f + oh2f, axis=0) / s
    pmean = jnp.sum(probs, axis=0) / s
    aux_ref[...] = jnp.reshape(LB_W * E * jnp.sum(f * pmean), (1, 1))


def _ffn_body(comb_ref, x_ref, wgu_ref, wd_ref, out_ref):
    e = pl.program_id(0)
    t = pl.program_id(1)

    @pl.when((e == 0) & (t == 0))
    def _():
        out_ref[...] = jnp.zeros_like(out_ref)

    x = x_ref[pl.ds(t * TT, TT), :]
    gu = jnp.dot(x, wgu_ref[0], preferred_element_type=jnp.float32)
    g = gu[:, :F]
    u = gu[:, F:]
    act = g * jax.nn.sigmoid(g) * u
    y = jnp.dot(act, wd_ref[0], preferred_element_type=jnp.float32)
    cvals = comb_ref[pl.ds(t * TT, TT), :]
    lane = jax.lax.broadcasted_iota(jnp.int32, cvals.shape, 1)
    scale = jnp.sum(jnp.where(lane == e, cvals, 0.0), axis=1, keepdims=True)
    out_ref[pl.ds(t * TT, TT), :] += scale * y


def kernel(x, gate_w, w_gate_up, w_down):
    b, s, h = x.shape
    x_flat = x.reshape(s, h)
    nt = s // TT

    comb, aux = pl.pallas_call(
        _gate_body,
        out_shape=[
            jax.ShapeDtypeStruct((s, E), jnp.float32),
            jax.ShapeDtypeStruct((1, 1), jnp.float32),
        ],
    )(x_flat, gate_w)

    out = pl.pallas_call(
        _ffn_body,
        grid=(E, nt),
        in_specs=[
            pl.BlockSpec((s, E), lambda e, t: (0, 0)),
            pl.BlockSpec((s, h), lambda e, t: (0, 0)),
            pl.BlockSpec((1, H, 2 * F), lambda e, t: (e, 0, 0)),
            pl.BlockSpec((1, F, H), lambda e, t: (e, 0, 0)),
        ],
        out_specs=pl.BlockSpec((s, h), lambda e, t: (0, 0)),
        out_shape=jax.ShapeDtypeStruct((s, h), jnp.float32),
    )(comb, x_flat, w_gate_up, w_down)

    return out.reshape(b, s, h), aux[0, 0]


# speedup vs baseline: 1.3508x; 1.0017x over previous
import jax
import jax.numpy as jnp
from jax.experimental import pallas as pl

E = 8
H = 768
F = 2048
TT = 256
LB_W = 0.01


def _gate_body(x_ref, gw_ref, comb_ref, aux_ref):
    x = x_ref[...]
    gw = gw_ref[...]
    logits = jax.lax.dot_general(
        x, gw, (((1,), (1,)), ((), ())), preferred_element_type=jnp.float32)
    m = jnp.max(logits, axis=-1, keepdims=True)
    ex = jnp.exp(logits - m)
    probs = ex / jnp.sum(ex, axis=-1, keepdims=True)

    iota = jax.lax.broadcasted_iota(jnp.int32, probs.shape, 1)
    p1 = jnp.max(probs, axis=-1, keepdims=True)
    idx1 = jnp.min(jnp.where(probs == p1, iota, E), axis=-1, keepdims=True)
    oh1 = (iota == idx1)
    masked = jnp.where(oh1, -jnp.inf, probs)
    p2 = jnp.max(masked, axis=-1, keepdims=True)
    idx2 = jnp.min(jnp.where(masked == p2, iota, E), axis=-1, keepdims=True)
    oh2 = (iota == idx2)

    denom = p1 + p2 + 1e-9
    oh1f = oh1.astype(jnp.float32)
    oh2f = oh2.astype(jnp.float32)
    comb_ref[...] = (p1 / denom) * oh1f + (p2 / denom) * oh2f

    s = jnp.float32(probs.shape[0])
    f = jnp.sum(oh1f + oh2f, axis=0) / s
    pmean = jnp.sum(probs, axis=0) / s
    aux_ref[...] = jnp.reshape(LB_W * E * jnp.sum(f * pmean), (1, 1))


def _ffn_body(comb_ref, x_ref, wgu_ref, wd_ref, out_ref):
    e = pl.program_id(0)
    t = pl.program_id(1)

    @pl.when((e == 0) & (t == 0))
    def _():
        out_ref[...] = jnp.zeros_like(out_ref)

    x = x_ref[pl.ds(t * TT, TT), :].astype(jnp.bfloat16)
    wgu = wgu_ref[0].astype(jnp.bfloat16)
    gu = jnp.dot(x, wgu, preferred_element_type=jnp.float32)
    g = gu[:, :F]
    u = gu[:, F:]
    act = (g * jax.nn.sigmoid(g) * u).astype(jnp.bfloat16)
    wd = wd_ref[0].astype(jnp.bfloat16)
    y = jnp.dot(act, wd, preferred_element_type=jnp.float32)
    cvals = comb_ref[pl.ds(t * TT, TT), :]
    lane = jax.lax.broadcasted_iota(jnp.int32, cvals.shape, 1)
    scale = jnp.sum(jnp.where(lane == e, cvals, 0.0), axis=1, keepdims=True)
    out_ref[pl.ds(t * TT, TT), :] += scale * y


def kernel(x, gate_w, w_gate_up, w_down):
    b, s, h = x.shape
    x_flat = x.reshape(s, h)
    nt = s // TT

    comb, aux = pl.pallas_call(
        _gate_body,
        out_shape=[
            jax.ShapeDtypeStruct((s, E), jnp.float32),
            jax.ShapeDtypeStruct((1, 1), jnp.float32),
        ],
    )(x_flat, gate_w)

    out = pl.pallas_call(
        _ffn_body,
        grid=(E, nt),
        in_specs=[
            pl.BlockSpec((s, E), lambda e, t: (0, 0)),
            pl.BlockSpec((s, h), lambda e, t: (0, 0)),
            pl.BlockSpec((1, H, 2 * F), lambda e, t: (e, 0, 0)),
            pl.BlockSpec((1, F, H), lambda e, t: (e, 0, 0)),
        ],
        out_specs=pl.BlockSpec((s, h), lambda e, t: (0, 0)),
        out_shape=jax.ShapeDtypeStruct((s, h), jnp.float32),
    )(comb, x_flat, w_gate_up, w_down)

    return out.reshape(b, s, h), aux[0, 0]
